# baseline (speedup 1.0000x reference)
import functools

import jax
import jax.numpy as jnp
from jax import lax
from jax.experimental import pallas as pl
from jax.experimental.pallas import tpu as pltpu
from jax.experimental.pallas import tpu_sc as plsc

N = 10000
NP = 10240
E = 320000
H = 128
K = 1544
TRASH = 10000
BM = 128
GM = 79
NT = 16
CPB = 16
NB = 10
NCH = CPB * NB
ECAP = NCH * 128
EP = NT * ECAP
EROW = EP // 128
ROWS_PER_TILE = NP // NT


def _leaky(v):
    return jnp.where(v > 0, v, 0.01 * v)


def _front_body(x_ref, wbig_ref, bbig_ref, win_ref, bin_ref, pa_ref,
                ws_ref, bs_ref, yy_ref):
    x = x_ref[...]
    h1 = jnp.dot(x, wbig_ref[...], preferred_element_type=jnp.float32)
    h1 = _leaky(h1 + bbig_ref[...])
    h = jnp.dot(h1, win_ref[...], preferred_element_type=jnp.float32)
    h = h + bin_ref[...]
    h = jnp.where(h > 0, h, pa_ref[...] * h)
    for t in range(3):
        yy_ref[t] = (
            jnp.dot(h, ws_ref[t], preferred_element_type=jnp.float32)
            + bs_ref[t])


def _combine(root, s, cnt):
    c0 = jnp.maximum(cnt[0, :, 0:1], 1.0)
    c1 = jnp.maximum(cnt[1, :, 0:1], 1.0)
    return root + s[0] / c0 + s[1] / c1


def _mid_body(yy_ref, s_ref, cnt_ref, ws_ref, bs_ref, yy2_ref):
    h = _combine(yy_ref[0], s_ref[...], cnt_ref[...])
    for t in range(3):
        yy2_ref[t] = (
            jnp.dot(h, ws_ref[t], preferred_element_type=jnp.float32)
            + bs_ref[t])


def _final_body(yy_ref, s_ref, cnt_ref, wcls_ref, bcls_ref, out_ref):
    h = _combine(yy_ref[0], s_ref[...], cnt_ref[...])
    out_ref[...] = (
        jnp.dot(h, wcls_ref[...], preferred_element_type=jnp.float32)
        + bcls_ref[...])


def _prep_body(src_ref, dst_ref, et_ref, p_ref, s_ref):
    s = src_ref[...]
    d = dst_ref[...]
    t = et_ref[...]
    trash = TRASH + lax.broadcasted_iota(jnp.int32, s.shape, 1)
    p_ref[...] = s + d * 16384
    s_ref[0] = jnp.where(t == 0, d, trash)
    s_ref[1] = jnp.where(t == 1, d, trash)


def _FULL(shape):
    return pl.BlockSpec(shape, lambda *_: tuple(0 for _ in shape))


def _YY3(rw=False):
    return pl.BlockSpec((3, BM, H), lambda i: (0, i, 0))


def _ROWB(nd=1):
    if nd == 1:
        return pl.BlockSpec((BM, H), lambda i: (i, 0))
    return pl.BlockSpec((2, BM, H), lambda i: (0, i, 0))


def _front(x, wbig, bbig, win, b_in, pa, wstack, bstack):
    return pl.pallas_call(
        _front_body,
        grid=(GM,),
        in_specs=[
            pl.BlockSpec((BM, K), lambda i: (i, 0)),
            _FULL((K, H)), _FULL((1, H)), _FULL((H, H)), _FULL((1, H)),
            _FULL((1, H)), _FULL((3, H, H)), _FULL((3, 1, H)),
        ],
        out_specs=_YY3(),
        out_shape=jax.ShapeDtypeStruct((3, NP, H), jnp.float32),
    )(x, wbig, bbig, win, b_in, pa, wstack, bstack)


def _mid(yy, s, cnt, wstack, bstack):
    return pl.pallas_call(
        _mid_body,
        grid=(GM,),
        in_specs=[
            _YY3(), _ROWB(2), _ROWB(2), _FULL((3, H, H)), _FULL((3, 1, H)),
        ],
        out_specs=_YY3(),
        out_shape=jax.ShapeDtypeStruct((3, NP, H), jnp.float32),
    )(yy, s, cnt, wstack, bstack)


def _final(yy, s, cnt, wcls, bcls):
    return pl.pallas_call(
        _final_body,
        grid=(GM,),
        in_specs=[_YY3(), _ROWB(2), _ROWB(2), _FULL((H, H)), _FULL((1, H))],
        out_specs=_ROWB(),
        out_shape=jax.ShapeDtypeStruct((N, H), jnp.float32),
    )(yy, s, cnt, wcls, bcls)


def _prep(src2d, dst2d, et2d):
    return pl.pallas_call(
        _prep_body,
        grid=(NCH,),
        in_specs=[pl.BlockSpec((EROW // NCH, 128), lambda i: (i, 0))] * 3,
        out_specs=[
            pl.BlockSpec((EROW // NCH, 128), lambda i: (i, 0)),
            pl.BlockSpec((2, EROW // NCH, 128), lambda i: (0, i, 0)),
        ],
        out_shape=[
            jax.ShapeDtypeStruct((EROW, 128), jnp.int32),
            jax.ShapeDtypeStruct((2, EROW, 128), jnp.int32),
        ],
    )(src2d, dst2d, et2d)


def _zero_buf(buf, rows):
    def _zrow(i, carry):
        for j in range(H // 16):
            buf[i, pl.ds(j * 16, 16)] = jnp.zeros((16,), jnp.float32)
        return carry
    lax.fori_loop(0, rows, _zrow, 0)


def _zero_acc(buf, acc, row0, rows=128):
    def _zcopy(kk, carry):
        pltpu.sync_copy(buf, acc.at[pl.ds(row0 + kk * rows, rows)])
        return carry
    lax.fori_loop(0, ROWS_PER_TILE // rows, _zcopy, 0)


def _write_out(buf, acc, out_hbm, c, row0, rows=128):
    def _obody(kk, carry):
        pltpu.sync_copy(acc.at[pl.ds(row0 + kk * rows, rows)], buf)
        pltpu.sync_copy(buf, out_hbm.at[c, pl.ds(row0 + kk * rows, rows)])
        return carry
    lax.fori_loop(0, ROWS_PER_TILE // rows, _obody, 0)


def _sc_cnt_body(pk_hbm, sidx_hbm, out_hbm, gidxc_hbm, sidxc_hbm,
                 cnt16_hbm, sidx_v, pk_v, list_v, ones_v, zbuf, cntv, acc):
    c = lax.axis_index("c")
    s = lax.axis_index("s")
    row0 = s * ROWS_PER_TILE
    erow0 = s * NB * CPB

    _zero_buf(zbuf, 16)
    e0 = jnp.where(lax.broadcasted_iota(jnp.int32, (16,), 0) == 0, 1.0, 0.0)

    def _orow(i, carry):
        ones_v[i, pl.ds(0, 16)] = e0
        for j in range(1, H // 16):
            ones_v[i, pl.ds(j * 16, 16)] = jnp.zeros((16,), jnp.float32)
        return carry
    lax.fori_loop(0, 128, _orow, 0)

    _zero_acc(zbuf, acc, row0, rows=16)

    lane = lax.broadcasted_iota(jnp.int32, (16,), 0)

    def _fill(i, carry):
        d = TRASH + ((i * 16 + lane) & 127)
        list_v[pl.ds(i * 16, 16)] = d * 16384
        return carry
    lax.fori_loop(0, ECAP // 16, _fill, 0)
    plsc.subcore_barrier()

    def _block(nb, off):
        pltpu.sync_copy(sidx_hbm.at[c, pl.ds(erow0 + nb * CPB, CPB)], sidx_v)
        pltpu.sync_copy(pk_hbm.at[pl.ds(erow0 + nb * CPB, CPB)], pk_v)

        def _row(r, off2):
            for l in range(8):
                sv = sidx_v[r, pl.ds(l * 16, 16)]
                pv = pk_v[r, pl.ds(l * 16, 16)]
                m = sv < TRASH
                pos = plsc.cumsum(jnp.where(m, 1, 0))
                idx = jnp.where(m, off2 + pos - 1,
                                ECAP + lax.broadcasted_iota(jnp.int32, (16,), 0))
                plsc.store_scatter(list_v, [idx], pv)
                off2 = off2 + plsc.all_reduce_population_count(m)
            return off2
        off = lax.fori_loop(0, CPB, _row, off)

        def _sct(j, carry):
            pltpu.sync_copy(ones_v, acc.at[sidx_v.at[j]], add=True)
            return carry
        lax.fori_loop(0, CPB, _sct, 0)
        return off
    off = lax.fori_loop(0, NB, _block, jnp.zeros((16,), jnp.int32))
    plsc.subcore_barrier()

    cntv[...] = off
    pltpu.sync_copy(cntv, cnt16_hbm.at[c, s])

    base = (1 + c) * NP

    def _ublk(nb, carry):
        def _urow(r, carry2):
            for l in range(8):
                pp = list_v[pl.ds(nb * 2048 + r * 128 + l * 16, 16)]
                d = lax.shift_right_logical(pp, 14)
                pk_v[r, pl.ds(l * 16, 16)] = pp - d * 16384 + base
                sidx_v[r, pl.ds(l * 16, 16)] = d
            return carry2
        lax.fori_loop(0, CPB, _urow, 0)
        pltpu.sync_copy(pk_v, gidxc_hbm.at[c, s, pl.ds(nb * CPB, CPB)])
        pltpu.sync_copy(sidx_v, sidxc_hbm.at[c, s, pl.ds(nb * CPB, CPB)])
        return carry
    lax.fori_loop(0, NB, _ublk, 0)
    _write_out(zbuf, acc, out_hbm, c, row0, rows=16)


def _sc_agg_body(yy_hbm, gidxc_hbm, sidxc_hbm, cnt16_hbm, out_hbm,
                 gidx_v, sidx_v, buf0, buf1, cntv, acc, sem):
    c = lax.axis_index("c")
    s = lax.axis_index("s")
    row0 = s * ROWS_PER_TILE

    _zero_buf(buf0, 128)
    _zero_acc(buf0, acc, row0)

    pltpu.sync_copy(cnt16_hbm.at[c, s], cntv)
    cnt = jnp.max(cntv[...])
    nblk = (cnt + (CPB * 128 - 1)) // (CPB * 128)
    nblk = 6
    plsc.subcore_barrier()

    def _block(nb, carry):
        pltpu.sync_copy(gidxc_hbm.at[c, s, pl.ds(nb * CPB, CPB)], gidx_v)
        pltpu.sync_copy(sidxc_hbm.at[c, s, pl.ds(nb * CPB, CPB)], sidx_v)
        pltpu.async_copy(yy_hbm.at[gidx_v.at[0]], buf0, sem)

        def _mbody(g, carry2):
            for b in range(2):
                bufa = buf0 if b == 0 else buf1
                bufb = buf1 if b == 0 else buf0
                j = g * 2 + b
                pltpu.make_async_copy(yy_hbm.at[gidx_v.at[j]], bufa,
                                      sem).wait()
                jn = jnp.minimum(j + 1, CPB - 1)
                pltpu.async_copy(yy_hbm.at[gidx_v.at[jn]], bufb, sem)
                pltpu.sync_copy(bufa, acc.at[sidx_v.at[j]], add=True)
            return carry2
        lax.fori_loop(0, CPB // 2, _mbody, 0)
        pltpu.make_async_copy(yy_hbm.at[gidx_v.at[0]], buf0, sem).wait()
        return carry
    lax.fori_loop(0, nblk, _block, 0)
    plsc.subcore_barrier()

    _write_out(buf1, acc, out_hbm, c, row0)


def _sc_mesh():
    return plsc.VectorSubcoreMesh(
        core_axis_name="c", subcore_axis_name="s",
        num_cores=2, num_subcores=NT)


@functools.cache
def _make_sc_cnt():
    return pl.kernel(
        _sc_cnt_body,
        out_type=(
            jax.ShapeDtypeStruct((2, NP, H), jnp.float32),
            jax.ShapeDtypeStruct((2, NT, NCH, 128), jnp.int32),
            jax.ShapeDtypeStruct((2, NT, NCH, 128), jnp.int32),
            jax.ShapeDtypeStruct((2, NT, 16), jnp.int32),
        ),
        mesh=_sc_mesh(),
        compiler_params=pltpu.CompilerParams(needs_layout_passes=False),
        scratch_types=[
            pltpu.VMEM((CPB, 128), jnp.int32),
            pltpu.VMEM((CPB, 128), jnp.int32),
            pltpu.VMEM((ECAP + 16,), jnp.int32),
            pltpu.VMEM((128, H), jnp.float32),
            pltpu.VMEM((16, H), jnp.float32),
            pltpu.VMEM((16,), jnp.int32),
            pltpu.VMEM_SHARED((NP, H), jnp.float32),
        ],
    )


@functools.cache
def _make_sc_agg():
    return pl.kernel(
        _sc_agg_body,
        out_type=jax.ShapeDtypeStruct((2, NP, H), jnp.float32),
        mesh=_sc_mesh(),
        compiler_params=pltpu.CompilerParams(needs_layout_passes=False),
        scratch_types=[
            pltpu.VMEM((CPB, 128), jnp.int32),
            pltpu.VMEM((CPB, 128), jnp.int32),
            pltpu.VMEM((128, H), jnp.float32),
            pltpu.VMEM((128, H), jnp.float32),
            pltpu.VMEM((16,), jnp.int32),
            pltpu.VMEM_SHARED((NP, H), jnp.float32),
            pltpu.SemaphoreType.DMA,
        ],
    )


def kernel(x, edge_index, edge_type, W_des, b_des, W_tweet, b_tweet,
           W_num, b_num, W_cat, b_cat, W_in, b_in, prelu_a,
           W_rel1, W_root1, b1, W_rel2, W_root2, b2, W_cls, b_cls):
    f32 = jnp.float32
    D_NUM, D_TWEET, D_CAT = 5, 768, 3

    wbig = jnp.zeros((K, H), f32)
    wbig = wbig.at[0:D_NUM, 64:96].set(W_num)
    wbig = wbig.at[D_NUM:D_NUM + D_TWEET, 32:64].set(W_tweet)
    wbig = wbig.at[D_NUM + D_TWEET:D_NUM + D_TWEET + D_CAT, 96:128].set(W_cat)
    wbig = wbig.at[D_NUM + D_TWEET + D_CAT:K, 0:32].set(W_des)
    bbig = jnp.concatenate([b_des, b_tweet, b_num, b_cat])[None, :]
    wstack1 = jnp.stack([W_root1, W_rel1[0], W_rel1[1]])
    bstack1 = jnp.stack([b1, jnp.zeros_like(b1), jnp.zeros_like(b1)])[:, None]
    wstack2 = jnp.stack([W_root2, W_rel2[0], W_rel2[1]])
    bstack2 = jnp.stack([b2, jnp.zeros_like(b2), jnp.zeros_like(b2)])[:, None]

    src = jnp.pad(edge_index[0].astype(jnp.int32), (0, EP - E))
    dst = jnp.pad(edge_index[1].astype(jnp.int32), (0, EP - E))
    et = jnp.pad(edge_type.astype(jnp.int32), (0, EP - E),
                 constant_values=2)
    packed, sidx = _prep(src.reshape(EROW, 128), dst.reshape(EROW, 128),
                         et.reshape(EROW, 128))

    cnts, gidxc, sidxc, cnt16 = _make_sc_cnt()(packed, sidx)
    yy1 = _front(x, wbig, bbig, W_in, b_in[None, :], prelu_a[None, :],
                 wstack1, bstack1)
    s1 = _make_sc_agg()(yy1.reshape(3 * NP, H), gidxc, sidxc, cnt16)
    yy2 = _mid(yy1, s1, cnts, wstack2, bstack2)
    s2 = _make_sc_agg()(yy2.reshape(3 * NP, H), gidxc, sidxc, cnt16)
    return _final(yy2, s2, cnts, W_cls, b_cls[None, :])


# speedup vs baseline: 5.1422x; 5.1422x over previous
import functools

import jax
import jax.numpy as jnp
from jax import lax
from jax.experimental import pallas as pl
from jax.experimental.pallas import tpu as pltpu
from jax.experimental.pallas import tpu_sc as plsc

N = 10000
NP = 10240
E = 320000
H = 128
K = 1544
TRASH = 10000
BM = 128
GM = 79
NT = 16
CPB = 16
NB = 10
NCH = CPB * NB
ECAP = NCH * 128
EP = NT * ECAP
EROW = EP // 128
ROWS_PER_TILE = NP // NT


def _leaky(v):
    return jnp.where(v > 0, v, 0.01 * v)


def _front_body(x_ref, wbig_ref, bbig_ref, win_ref, bin_ref, pa_ref,
                ws_ref, bs_ref, yy_ref):
    x = x_ref[...]
    h1 = jnp.dot(x, wbig_ref[...], preferred_element_type=jnp.float32)
    h1 = _leaky(h1 + bbig_ref[...])
    h = jnp.dot(h1, win_ref[...], preferred_element_type=jnp.float32)
    h = h + bin_ref[...]
    h = jnp.where(h > 0, h, pa_ref[...] * h)
    for t in range(3):
        yy_ref[t] = (
            jnp.dot(h, ws_ref[t], preferred_element_type=jnp.float32)
            + bs_ref[t])


def _combine(root, s, cnt):
    c0 = jnp.maximum(cnt[0, :, 0:1], 1.0)
    c1 = jnp.maximum(cnt[1, :, 0:1], 1.0)
    return root + s[0] / c0 + s[1] / c1


def _mid_body(yy_ref, s_ref, cnt_ref, ws_ref, bs_ref, yy2_ref):
    h = _combine(yy_ref[0], s_ref[...], cnt_ref[...])
    for t in range(3):
        yy2_ref[t] = (
            jnp.dot(h, ws_ref[t], preferred_element_type=jnp.float32)
            + bs_ref[t])


def _final_body(yy_ref, s_ref, cnt_ref, wcls_ref, bcls_ref, out_ref):
    h = _combine(yy_ref[0], s_ref[...], cnt_ref[...])
    out_ref[...] = (
        jnp.dot(h, wcls_ref[...], preferred_element_type=jnp.float32)
        + bcls_ref[...])


def _prep_body(src_ref, dst_ref, et_ref, p_ref, s_ref):
    s = src_ref[...]
    d = dst_ref[...]
    t = et_ref[...]
    trash = TRASH + lax.broadcasted_iota(jnp.int32, s.shape, 1)
    p_ref[...] = s + d * 16384
    s_ref[0] = jnp.where(t == 0, d, trash)
    s_ref[1] = jnp.where(t == 1, d, trash)


def _FULL(shape):
    return pl.BlockSpec(shape, lambda *_: tuple(0 for _ in shape))


def _YY3(rw=False):
    return pl.BlockSpec((3, BM, H), lambda i: (0, i, 0))


def _ROWB(nd=1):
    if nd == 1:
        return pl.BlockSpec((BM, H), lambda i: (i, 0))
    return pl.BlockSpec((2, BM, H), lambda i: (0, i, 0))


def _front(x, wbig, bbig, win, b_in, pa, wstack, bstack):
    return pl.pallas_call(
        _front_body,
        grid=(GM,),
        in_specs=[
            pl.BlockSpec((BM, K), lambda i: (i, 0)),
            _FULL((K, H)), _FULL((1, H)), _FULL((H, H)), _FULL((1, H)),
            _FULL((1, H)), _FULL((3, H, H)), _FULL((3, 1, H)),
        ],
        out_specs=_YY3(),
        out_shape=jax.ShapeDtypeStruct((3, NP, H), jnp.float32),
    )(x, wbig, bbig, win, b_in, pa, wstack, bstack)


def _mid(yy, s, cnt, wstack, bstack):
    return pl.pallas_call(
        _mid_body,
        grid=(GM,),
        in_specs=[
            _YY3(), _ROWB(2), _ROWB(2), _FULL((3, H, H)), _FULL((3, 1, H)),
        ],
        out_specs=_YY3(),
        out_shape=jax.ShapeDtypeStruct((3, NP, H), jnp.float32),
    )(yy, s, cnt, wstack, bstack)


def _final(yy, s, cnt, wcls, bcls):
    return pl.pallas_call(
        _final_body,
        grid=(GM,),
        in_specs=[_YY3(), _ROWB(2), _ROWB(2), _FULL((H, H)), _FULL((1, H))],
        out_specs=_ROWB(),
        out_shape=jax.ShapeDtypeStruct((N, H), jnp.float32),
    )(yy, s, cnt, wcls, bcls)


def _prep(src2d, dst2d, et2d):
    return pl.pallas_call(
        _prep_body,
        grid=(NCH,),
        in_specs=[pl.BlockSpec((EROW // NCH, 128), lambda i: (i, 0))] * 3,
        out_specs=[
            pl.BlockSpec((EROW // NCH, 128), lambda i: (i, 0)),
            pl.BlockSpec((2, EROW // NCH, 128), lambda i: (0, i, 0)),
        ],
        out_shape=[
            jax.ShapeDtypeStruct((EROW, 128), jnp.int32),
            jax.ShapeDtypeStruct((2, EROW, 128), jnp.int32),
        ],
    )(src2d, dst2d, et2d)


def _zero_buf(buf, rows):
    def _zrow(i, carry):
        for j in range(H // 16):
            buf[i, pl.ds(j * 16, 16)] = jnp.zeros((16,), jnp.float32)
        return carry
    lax.fori_loop(0, rows, _zrow, 0)


def _zero_acc(buf, acc, row0, rows=128):
    def _zcopy(kk, carry):
        pltpu.sync_copy(buf, acc.at[pl.ds(row0 + kk * rows, rows)])
        return carry
    lax.fori_loop(0, ROWS_PER_TILE // rows, _zcopy, 0)


def _write_out(buf, acc, out_hbm, c, row0, rows=128):
    def _obody(kk, carry):
        pltpu.sync_copy(acc.at[pl.ds(row0 + kk * rows, rows)], buf)
        pltpu.sync_copy(buf, out_hbm.at[c, pl.ds(row0 + kk * rows, rows)])
        return carry
    lax.fori_loop(0, ROWS_PER_TILE // rows, _obody, 0)


def _sc_cnt_body(pk_hbm, sidx_hbm, out_hbm, gidxc_hbm, sidxc_hbm,
                 cnt16_hbm, sidx_v, pk_v, list_v, ones_v, zbuf, cntv, acc):
    c = lax.axis_index("c")
    s = lax.axis_index("s")
    row0 = s * ROWS_PER_TILE
    erow0 = s * NB * CPB

    _zero_buf(zbuf, 16)
    e0 = jnp.where(lax.broadcasted_iota(jnp.int32, (16,), 0) == 0, 1.0, 0.0)

    def _orow(i, carry):
        ones_v[i, pl.ds(0, 16)] = e0
        for j in range(1, H // 16):
            ones_v[i, pl.ds(j * 16, 16)] = jnp.zeros((16,), jnp.float32)
        return carry
    lax.fori_loop(0, 128, _orow, 0)

    _zero_acc(zbuf, acc, row0, rows=16)

    lane = lax.broadcasted_iota(jnp.int32, (16,), 0)

    def _fill(i, carry):
        d = TRASH + s * 15 + ((i * 16 + lane) % 15)
        list_v[pl.ds(i * 16, 16)] = d * 16384
        return carry
    lax.fori_loop(0, ECAP // 16, _fill, 0)
    plsc.subcore_barrier()

    def _block(nb, off):
        pltpu.sync_copy(sidx_hbm.at[c, pl.ds(erow0 + nb * CPB, CPB)], sidx_v)
        pltpu.sync_copy(pk_hbm.at[pl.ds(erow0 + nb * CPB, CPB)], pk_v)

        def _row(r, off2):
            for l in range(8):
                sv = sidx_v[r, pl.ds(l * 16, 16)]
                pv = pk_v[r, pl.ds(l * 16, 16)]
                m = sv < TRASH
                pos = plsc.cumsum(jnp.where(m, 1, 0))
                idx = jnp.where(m, off2 + pos - 1,
                                ECAP + lax.broadcasted_iota(jnp.int32, (16,), 0))
                plsc.store_scatter(list_v, [idx], pv)
                off2 = off2 + plsc.all_reduce_population_count(m)
            return off2
        off = lax.fori_loop(0, CPB, _row, off)
        return off
    off = lax.fori_loop(0, NB, _block, jnp.zeros((16,), jnp.int32))
    plsc.subcore_barrier()

    cntv[...] = off
    pltpu.sync_copy(cntv, cnt16_hbm.at[c, s])

    base = (1 + c) * NP
    ncht = (jnp.max(off) + 127) // 128

    def _ublk(nb, carry):
        def _urow(r, carry2):
            for l in range(8):
                pp = list_v[pl.ds(nb * 2048 + r * 128 + l * 16, 16)]
                d = lax.shift_right_logical(pp, 14)
                pk_v[r, pl.ds(l * 16, 16)] = pp - d * 16384 + base
                sidx_v[r, pl.ds(l * 16, 16)] = d
            return carry2
        lax.fori_loop(0, CPB, _urow, 0)
        pltpu.sync_copy(pk_v, gidxc_hbm.at[c, s, pl.ds(nb * CPB, CPB)])
        pltpu.sync_copy(sidx_v, sidxc_hbm.at[c, s, pl.ds(nb * CPB, CPB)])

        def _sct(j, carry2):
            @pl.when(nb * CPB + j < ncht)
            def _():
                pltpu.sync_copy(ones_v, acc.at[sidx_v.at[j]], add=True)
            return carry2
        lax.fori_loop(0, CPB, _sct, 0)
        return carry
    lax.fori_loop(0, NB, _ublk, 0)
    _write_out(zbuf, acc, out_hbm, c, row0, rows=16)


def _sc_agg_body(yy_hbm, gidxc_hbm, sidxc_hbm, cnt16_hbm, out_hbm,
                 gidx_v, sidx_v, buf0, buf1, cntv, acc, sem):
    c = lax.axis_index("c")
    s = lax.axis_index("s")
    row0 = s * ROWS_PER_TILE

    _zero_buf(buf0, 128)
    _zero_acc(buf0, acc, row0)

    pltpu.sync_copy(cnt16_hbm.at[c, s], cntv)
    cnt = jnp.max(cntv[...])
    nfull = cnt // (CPB * 128)
    nrem = (cnt - nfull * (CPB * 128) + 127) // 128
    plsc.subcore_barrier()

    def _block(nb, carry):
        pltpu.sync_copy(gidxc_hbm.at[c, s, pl.ds(nb * CPB, CPB)], gidx_v)
        pltpu.sync_copy(sidxc_hbm.at[c, s, pl.ds(nb * CPB, CPB)], sidx_v)
        pltpu.async_copy(yy_hbm.at[gidx_v.at[0]], buf0, sem)

        def _mbody(g, carry2):
            for b in range(2):
                bufa = buf0 if b == 0 else buf1
                bufb = buf1 if b == 0 else buf0
                j = g * 2 + b
                pltpu.make_async_copy(yy_hbm.at[gidx_v.at[j]], bufa,
                                      sem).wait()
                jn = jnp.minimum(j + 1, CPB - 1)
                pltpu.async_copy(yy_hbm.at[gidx_v.at[jn]], bufb, sem)
                pltpu.sync_copy(bufa, acc.at[sidx_v.at[j]], add=True)
            return carry2
        lax.fori_loop(0, CPB // 2, _mbody, 0)
        pltpu.make_async_copy(yy_hbm.at[gidx_v.at[0]], buf0, sem).wait()
        return carry
    lax.fori_loop(0, nfull, _block, 0)

    @pl.when(nrem > 0)
    def _():
        pltpu.sync_copy(gidxc_hbm.at[c, s, pl.ds(nfull * CPB, CPB)], gidx_v)
        pltpu.sync_copy(sidxc_hbm.at[c, s, pl.ds(nfull * CPB, CPB)], sidx_v)

        def _rchunk(j, carry):
            pltpu.async_copy(yy_hbm.at[gidx_v.at[j]], buf0, sem).wait()
            pltpu.sync_copy(buf0, acc.at[sidx_v.at[j]], add=True)
            return carry
        lax.fori_loop(0, nrem, _rchunk, 0)
    plsc.subcore_barrier()

    _write_out(buf1, acc, out_hbm, c, row0)


def _sc_mesh():
    return plsc.VectorSubcoreMesh(
        core_axis_name="c", subcore_axis_name="s",
        num_cores=2, num_subcores=NT)


@functools.cache
def _make_sc_cnt():
    return pl.kernel(
        _sc_cnt_body,
        out_type=(
            jax.ShapeDtypeStruct((2, NP, H), jnp.float32),
            jax.ShapeDtypeStruct((2, NT, NCH, 128), jnp.int32),
            jax.ShapeDtypeStruct((2, NT, NCH, 128), jnp.int32),
            jax.ShapeDtypeStruct((2, NT, 16), jnp.int32),
        ),
        mesh=_sc_mesh(),
        compiler_params=pltpu.CompilerParams(needs_layout_passes=False),
        scratch_types=[
            pltpu.VMEM((CPB, 128), jnp.int32),
            pltpu.VMEM((CPB, 128), jnp.int32),
            pltpu.VMEM((ECAP + 16,), jnp.int32),
            pltpu.VMEM((128, H), jnp.float32),
            pltpu.VMEM((16, H), jnp.float32),
            pltpu.VMEM((16,), jnp.int32),
            pltpu.VMEM_SHARED((NP, H), jnp.float32),
        ],
    )


@functools.cache
def _make_sc_agg():
    return pl.kernel(
        _sc_agg_body,
        out_type=jax.ShapeDtypeStruct((2, NP, H), jnp.float32),
        mesh=_sc_mesh(),
        compiler_params=pltpu.CompilerParams(needs_layout_passes=False),
        scratch_types=[
            pltpu.VMEM((CPB, 128), jnp.int32),
            pltpu.VMEM((CPB, 128), jnp.int32),
            pltpu.VMEM((128, H), jnp.float32),
            pltpu.VMEM((128, H), jnp.float32),
            pltpu.VMEM((16,), jnp.int32),
            pltpu.VMEM_SHARED((NP, H), jnp.float32),
            pltpu.SemaphoreType.DMA,
        ],
    )


def kernel(x, edge_index, edge_type, W_des, b_des, W_tweet, b_tweet,
           W_num, b_num, W_cat, b_cat, W_in, b_in, prelu_a,
           W_rel1, W_root1, b1, W_rel2, W_root2, b2, W_cls, b_cls):
    f32 = jnp.float32
    D_NUM, D_TWEET, D_CAT = 5, 768, 3

    wbig = jnp.zeros((K, H), f32)
    wbig = wbig.at[0:D_NUM, 64:96].set(W_num)
    wbig = wbig.at[D_NUM:D_NUM + D_TWEET, 32:64].set(W_tweet)
    wbig = wbig.at[D_NUM + D_TWEET:D_NUM + D_TWEET + D_CAT, 96:128].set(W_cat)
    wbig = wbig.at[D_NUM + D_TWEET + D_CAT:K, 0:32].set(W_des)
    bbig = jnp.concatenate([b_des, b_tweet, b_num, b_cat])[None, :]
    wstack1 = jnp.stack([W_root1, W_rel1[0], W_rel1[1]])
    bstack1 = jnp.stack([b1, jnp.zeros_like(b1), jnp.zeros_like(b1)])[:, None]
    wstack2 = jnp.stack([W_root2, W_rel2[0], W_rel2[1]])
    bstack2 = jnp.stack([b2, jnp.zeros_like(b2), jnp.zeros_like(b2)])[:, None]

    src = jnp.pad(edge_index[0].astype(jnp.int32), (0, EP - E))
    dst = jnp.pad(edge_index[1].astype(jnp.int32), (0, EP - E))
    et = jnp.pad(edge_type.astype(jnp.int32), (0, EP - E),
                 constant_values=2)
    packed, sidx = _prep(src.reshape(EROW, 128), dst.reshape(EROW, 128),
                         et.reshape(EROW, 128))

    cnts, gidxc, sidxc, cnt16 = _make_sc_cnt()(packed, sidx)
    yy1 = _front(x, wbig, bbig, W_in, b_in[None, :], prelu_a[None, :],
                 wstack1, bstack1)
    s1 = _make_sc_agg()(yy1.reshape(3 * NP, H), gidxc, sidxc, cnt16)
    yy2 = _mid(yy1, s1, cnts, wstack2, bstack2)
    s2 = _make_sc_agg()(yy2.reshape(3 * NP, H), gidxc, sidxc, cnt16)
    return _final(yy2, s2, cnts, W_cls, b_cls[None, :])


# speedup vs baseline: 5.1509x; 1.0017x over previous
import functools

import jax
import jax.numpy as jnp
from jax import lax
from jax.experimental import pallas as pl
from jax.experimental.pallas import tpu as pltpu
from jax.experimental.pallas import tpu_sc as plsc

N = 10000
NP = 10240
E = 320000
H = 128
K = 1544
TRASH = 10000
BM = 128
GM = 79
NT = 16
CPB = 16
NB = 10
NCH = CPB * NB
ECAP = NCH * 128
EP = NT * ECAP
EROW = EP // 128
ROWS_PER_TILE = NP // NT


def _leaky(v):
    return jnp.where(v > 0, v, 0.01 * v)


def _front_body(x_ref, wbig_ref, bbig_ref, win_ref, bin_ref, pa_ref,
                ws_ref, bs_ref, yy_ref):
    x = x_ref[...]
    h1 = jnp.dot(x, wbig_ref[...], preferred_element_type=jnp.float32)
    h1 = _leaky(h1 + bbig_ref[...])
    h = jnp.dot(h1, win_ref[...], preferred_element_type=jnp.float32)
    h = h + bin_ref[...]
    h = jnp.where(h > 0, h, pa_ref[...] * h)
    for t in range(3):
        yy_ref[t] = (
            jnp.dot(h, ws_ref[t], preferred_element_type=jnp.float32)
            + bs_ref[t])


def _combine(root, s, cnt):
    c0 = jnp.maximum(cnt[0, :, 0:1], 1.0)
    c1 = jnp.maximum(cnt[1, :, 0:1], 1.0)
    return root + s[0] / c0 + s[1] / c1


def _mid_body(yy_ref, s_ref, cnt_ref, ws_ref, bs_ref, yy2_ref):
    h = _combine(yy_ref[0], s_ref[...], cnt_ref[...])
    for t in range(3):
        yy2_ref[t] = (
            jnp.dot(h, ws_ref[t], preferred_element_type=jnp.float32)
            + bs_ref[t])


def _final_body(yy_ref, s_ref, cnt_ref, wcls_ref, bcls_ref, out_ref):
    h = _combine(yy_ref[0], s_ref[...], cnt_ref[...])
    out_ref[...] = (
        jnp.dot(h, wcls_ref[...], preferred_element_type=jnp.float32)
        + bcls_ref[...])


def _prep_body(src_ref, dst_ref, et_ref, p_ref, s_ref):
    s = src_ref[...]
    d = dst_ref[...]
    t = et_ref[...]
    trash = TRASH + lax.broadcasted_iota(jnp.int32, s.shape, 1)
    p_ref[...] = s + d * 16384
    s_ref[0] = jnp.where(t == 0, d, trash)
    s_ref[1] = jnp.where(t == 1, d, trash)


def _FULL(shape):
    return pl.BlockSpec(shape, lambda *_: tuple(0 for _ in shape))


def _YY3(rw=False):
    return pl.BlockSpec((3, BM, H), lambda i: (0, i, 0))


def _ROWB(nd=1):
    if nd == 1:
        return pl.BlockSpec((BM, H), lambda i: (i, 0))
    return pl.BlockSpec((2, BM, H), lambda i: (0, i, 0))


def _front(x, wbig, bbig, win, b_in, pa, wstack, bstack):
    return pl.pallas_call(
        _front_body,
        grid=(GM,),
        in_specs=[
            pl.BlockSpec((BM, K), lambda i: (i, 0)),
            _FULL((K, H)), _FULL((1, H)), _FULL((H, H)), _FULL((1, H)),
            _FULL((1, H)), _FULL((3, H, H)), _FULL((3, 1, H)),
        ],
        out_specs=_YY3(),
        out_shape=jax.ShapeDtypeStruct((3, NP, H), jnp.float32),
    )(x, wbig, bbig, win, b_in, pa, wstack, bstack)


def _mid(yy, s, cnt, wstack, bstack):
    return pl.pallas_call(
        _mid_body,
        grid=(GM,),
        in_specs=[
            _YY3(), _ROWB(2), _ROWB(2), _FULL((3, H, H)), _FULL((3, 1, H)),
        ],
        out_specs=_YY3(),
        out_shape=jax.ShapeDtypeStruct((3, NP, H), jnp.float32),
    )(yy, s, cnt, wstack, bstack)


def _final(yy, s, cnt, wcls, bcls):
    return pl.pallas_call(
        _final_body,
        grid=(GM,),
        in_specs=[_YY3(), _ROWB(2), _ROWB(2), _FULL((H, H)), _FULL((1, H))],
        out_specs=_ROWB(),
        out_shape=jax.ShapeDtypeStruct((N, H), jnp.float32),
    )(yy, s, cnt, wcls, bcls)


def _prep(src2d, dst2d, et2d):
    return pl.pallas_call(
        _prep_body,
        grid=(NCH,),
        in_specs=[pl.BlockSpec((EROW // NCH, 128), lambda i: (i, 0))] * 3,
        out_specs=[
            pl.BlockSpec((EROW // NCH, 128), lambda i: (i, 0)),
            pl.BlockSpec((2, EROW // NCH, 128), lambda i: (0, i, 0)),
        ],
        out_shape=[
            jax.ShapeDtypeStruct((EROW, 128), jnp.int32),
            jax.ShapeDtypeStruct((2, EROW, 128), jnp.int32),
        ],
    )(src2d, dst2d, et2d)


def _zero_buf(buf, rows):
    def _zrow(i, carry):
        for j in range(H // 16):
            buf[i, pl.ds(j * 16, 16)] = jnp.zeros((16,), jnp.float32)
        return carry
    lax.fori_loop(0, rows, _zrow, 0)


def _zero_acc(buf, acc, row0, rows=128):
    def _zcopy(kk, carry):
        pltpu.sync_copy(buf, acc.at[pl.ds(row0 + kk * rows, rows)])
        return carry
    lax.fori_loop(0, ROWS_PER_TILE // rows, _zcopy, 0)


def _write_out(buf, acc, out_hbm, c, row0, rows=128):
    def _obody(kk, carry):
        pltpu.sync_copy(acc.at[pl.ds(row0 + kk * rows, rows)], buf)
        pltpu.sync_copy(buf, out_hbm.at[c, pl.ds(row0 + kk * rows, rows)])
        return carry
    lax.fori_loop(0, ROWS_PER_TILE // rows, _obody, 0)


def _sc_cnt_body(pk_hbm, sidx_hbm, out_hbm, gidxc_hbm, sidxc_hbm,
                 cnt16_hbm, sidx_v, pk_v, list_v, ones_v, zbuf, cntv, acc):
    c = lax.axis_index("c")
    s = lax.axis_index("s")
    row0 = s * ROWS_PER_TILE
    erow0 = s * NB * CPB

    _zero_buf(zbuf, 16)
    e0 = jnp.where(lax.broadcasted_iota(jnp.int32, (16,), 0) == 0, 1.0, 0.0)

    def _orow(i, carry):
        ones_v[i, pl.ds(0, 16)] = e0
        for j in range(1, H // 16):
            ones_v[i, pl.ds(j * 16, 16)] = jnp.zeros((16,), jnp.float32)
        return carry
    lax.fori_loop(0, 128, _orow, 0)

    _zero_acc(zbuf, acc, row0, rows=16)

    lane = lax.broadcasted_iota(jnp.int32, (16,), 0)

    def _fill(i, carry):
        d = TRASH + s * 15 + ((i * 16 + lane) % 15)
        list_v[pl.ds(i * 16, 16)] = d * 16384
        return carry
    lax.fori_loop(0, ECAP // 16, _fill, 0)
    plsc.subcore_barrier()

    def _block(nb, off):
        pltpu.sync_copy(sidx_hbm.at[c, pl.ds(erow0 + nb * CPB, CPB)], sidx_v)
        pltpu.sync_copy(pk_hbm.at[pl.ds(erow0 + nb * CPB, CPB)], pk_v)

        def _row(r, off2):
            for l in range(8):
                sv = sidx_v[r, pl.ds(l * 16, 16)]
                pv = pk_v[r, pl.ds(l * 16, 16)]
                m = sv < TRASH
                pos = plsc.cumsum(jnp.where(m, 1, 0))
                idx = jnp.where(m, off2 + pos - 1,
                                ECAP + lax.broadcasted_iota(jnp.int32, (16,), 0))
                plsc.store_scatter(list_v, [idx], pv)
                off2 = off2 + plsc.all_reduce_population_count(m)
            return off2
        off = lax.fori_loop(0, CPB, _row, off)
        return off
    off = lax.fori_loop(0, NB, _block, jnp.zeros((16,), jnp.int32))
    plsc.subcore_barrier()

    cntv[...] = off
    pltpu.sync_copy(cntv, cnt16_hbm.at[c, s])

    base = (1 + c) * NP
    ncht = (jnp.max(cntv[...]) + 127) // 128

    def _ublk(nb, carry):
        def _urow(r, carry2):
            for l in range(8):
                pp = list_v[pl.ds(nb * 2048 + r * 128 + l * 16, 16)]
                d = lax.shift_right_logical(pp, 14)
                pk_v[r, pl.ds(l * 16, 16)] = pp - d * 16384 + base
                sidx_v[r, pl.ds(l * 16, 16)] = d
            return carry2
        lax.fori_loop(0, CPB, _urow, 0)
        pltpu.sync_copy(pk_v, gidxc_hbm.at[c, s, pl.ds(nb * CPB, CPB)])
        pltpu.sync_copy(sidx_v, sidxc_hbm.at[c, s, pl.ds(nb * CPB, CPB)])

        def _sct(j, carry2):
            @pl.when(nb * CPB + j < ncht)
            def _():
                pltpu.sync_copy(ones_v, acc.at[sidx_v.at[j]], add=True)
            return carry2
        lax.fori_loop(0, CPB, _sct, 0)
        return carry
    lax.fori_loop(0, NB, _ublk, 0)

    plsc.subcore_barrier()
    _write_out(zbuf, acc, out_hbm, c, row0, rows=16)


def _sc_agg_body(yy_hbm, gidxc_hbm, sidxc_hbm, cnt16_hbm, out_hbm,
                 gidx_v, sidx_v, buf0, buf1, cntv, acc, sem):
    c = lax.axis_index("c")
    s = lax.axis_index("s")
    row0 = s * ROWS_PER_TILE

    _zero_buf(buf0, 128)
    _zero_acc(buf0, acc, row0)

    pltpu.sync_copy(cnt16_hbm.at[c, s], cntv)
    cnt = jnp.max(cntv[...])
    nfull = cnt // (CPB * 128)
    nrem = (cnt - nfull * (CPB * 128) + 127) // 128
    plsc.subcore_barrier()

    def _block(nb, carry):
        pltpu.sync_copy(gidxc_hbm.at[c, s, pl.ds(nb * CPB, CPB)], gidx_v)
        pltpu.sync_copy(sidxc_hbm.at[c, s, pl.ds(nb * CPB, CPB)], sidx_v)
        pltpu.async_copy(yy_hbm.at[gidx_v.at[0]], buf0, sem)

        def _mbody(g, carry2):
            for b in range(2):
                bufa = buf0 if b == 0 else buf1
                bufb = buf1 if b == 0 else buf0
                j = g * 2 + b
                pltpu.make_async_copy(yy_hbm.at[gidx_v.at[j]], bufa,
                                      sem).wait()
                jn = jnp.minimum(j + 1, CPB - 1)
                pltpu.async_copy(yy_hbm.at[gidx_v.at[jn]], bufb, sem)
                pltpu.sync_copy(bufa, acc.at[sidx_v.at[j]], add=True)
            return carry2
        lax.fori_loop(0, CPB // 2, _mbody, 0)
        pltpu.make_async_copy(yy_hbm.at[gidx_v.at[0]], buf0, sem).wait()
        return carry
    lax.fori_loop(0, nfull, _block, 0)

    @pl.when(nrem > 0)
    def _():
        pltpu.sync_copy(gidxc_hbm.at[c, s, pl.ds(nfull * CPB, CPB)], gidx_v)
        pltpu.sync_copy(sidxc_hbm.at[c, s, pl.ds(nfull * CPB, CPB)], sidx_v)

        def _rchunk(j, carry):
            pltpu.async_copy(yy_hbm.at[gidx_v.at[j]], buf0, sem).wait()
            pltpu.sync_copy(buf0, acc.at[sidx_v.at[j]], add=True)
            return carry
        lax.fori_loop(0, nrem, _rchunk, 0)
    plsc.subcore_barrier()

    _write_out(buf1, acc, out_hbm, c, row0)


def _sc_mesh():
    return plsc.VectorSubcoreMesh(
        core_axis_name="c", subcore_axis_name="s",
        num_cores=2, num_subcores=NT)


@functools.cache
def _make_sc_cnt():
    return pl.kernel(
        _sc_cnt_body,
        out_type=(
            jax.ShapeDtypeStruct((2, NP, H), jnp.float32),
            jax.ShapeDtypeStruct((2, NT, NCH, 128), jnp.int32),
            jax.ShapeDtypeStruct((2, NT, NCH, 128), jnp.int32),
            jax.ShapeDtypeStruct((2, NT, 16), jnp.int32),
        ),
        mesh=_sc_mesh(),
        compiler_params=pltpu.CompilerParams(needs_layout_passes=False),
        scratch_types=[
            pltpu.VMEM((CPB, 128), jnp.int32),
            pltpu.VMEM((CPB, 128), jnp.int32),
            pltpu.VMEM((ECAP + 16,), jnp.int32),
            pltpu.VMEM((128, H), jnp.float32),
            pltpu.VMEM((16, H), jnp.float32),
            pltpu.VMEM((16,), jnp.int32),
            pltpu.VMEM_SHARED((NP, H), jnp.float32),
        ],
    )


@functools.cache
def _make_sc_agg():
    return pl.kernel(
        _sc_agg_body,
        out_type=jax.ShapeDtypeStruct((2, NP, H), jnp.float32),
        mesh=_sc_mesh(),
        compiler_params=pltpu.CompilerParams(needs_layout_passes=False),
        scratch_types=[
            pltpu.VMEM((CPB, 128), jnp.int32),
            pltpu.VMEM((CPB, 128), jnp.int32),
            pltpu.VMEM((128, H), jnp.float32),
            pltpu.VMEM((128, H), jnp.float32),
            pltpu.VMEM((16,), jnp.int32),
            pltpu.VMEM_SHARED((NP, H), jnp.float32),
            pltpu.SemaphoreType.DMA,
        ],
    )


def kernel(x, edge_index, edge_type, W_des, b_des, W_tweet, b_tweet,
           W_num, b_num, W_cat, b_cat, W_in, b_in, prelu_a,
           W_rel1, W_root1, b1, W_rel2, W_root2, b2, W_cls, b_cls):
    f32 = jnp.float32
    D_NUM, D_TWEET, D_CAT = 5, 768, 3

    wbig = jnp.zeros((K, H), f32)
    wbig = wbig.at[0:D_NUM, 64:96].set(W_num)
    wbig = wbig.at[D_NUM:D_NUM + D_TWEET, 32:64].set(W_tweet)
    wbig = wbig.at[D_NUM + D_TWEET:D_NUM + D_TWEET + D_CAT, 96:128].set(W_cat)
    wbig = wbig.at[D_NUM + D_TWEET + D_CAT:K, 0:32].set(W_des)
    bbig = jnp.concatenate([b_des, b_tweet, b_num, b_cat])[None, :]
    wstack1 = jnp.stack([W_root1, W_rel1[0], W_rel1[1]])
    bstack1 = jnp.stack([b1, jnp.zeros_like(b1), jnp.zeros_like(b1)])[:, None]
    wstack2 = jnp.stack([W_root2, W_rel2[0], W_rel2[1]])
    bstack2 = jnp.stack([b2, jnp.zeros_like(b2), jnp.zeros_like(b2)])[:, None]

    src = jnp.pad(edge_index[0].astype(jnp.int32), (0, EP - E))
    dst = jnp.pad(edge_index[1].astype(jnp.int32), (0, EP - E))
    et = jnp.pad(edge_type.astype(jnp.int32), (0, EP - E),
                 constant_values=2)
    packed, sidx = _prep(src.reshape(EROW, 128), dst.reshape(EROW, 128),
                         et.reshape(EROW, 128))

    cnts, gidxc, sidxc, cnt16 = _make_sc_cnt()(packed, sidx)
    yy1 = _front(x, wbig, bbig, W_in, b_in[None, :], prelu_a[None, :],
                 wstack1, bstack1)
    s1 = _make_sc_agg()(yy1.reshape(3 * NP, H), gidxc, sidxc, cnt16)
    yy2 = _mid(yy1, s1, cnts, wstack2, bstack2)
    s2 = _make_sc_agg()(yy2.reshape(3 * NP, H), gidxc, sidxc, cnt16)
    return _final(yy2, s2, cnts, W_cls, b_cls[None, :])
